# Initial kernel scaffold; baseline (speedup 1.0000x reference)
import jax
import jax.numpy as jnp
from jax.experimental import pallas as pl


def kernel(x, segment_ids, token_table, segment_table, position_table):
    raise NotImplementedError("write your pallas kernel here")


# speedup vs baseline: 2.9768x; 2.9768x over previous
import functools
import jax
import jax.numpy as jnp
from jax import lax
from jax.experimental import pallas as pl
from jax.experimental.pallas import tpu as pltpu
from jax.experimental.pallas import tpu_sc as plsc

_HIDDEN = 128
_MAXLEN = 200
_LANES = 16
_NCORES = 2
_NSUB = 16
_NWORK = _NCORES * _NSUB
_SEQ_PER_STEP = 2
_R = _SEQ_PER_STEP * _MAXLEN


def _body(x_ref, seg_ref, tok_ref, segtab_ref, postab_ref, out_ref,
          tok_v, comb0_v, idx_v, sgv_v, segtab_v, sem):
    n_rows = x_ref.shape[0]
    rows_per_w = n_rows // _NWORK
    steps = rows_per_w // _R

    wid = lax.axis_index("s") * _NCORES + lax.axis_index("c")

    pltpu.sync_copy(postab_ref, comb0_v)
    pltpu.sync_copy(segtab_ref, segtab_v)

    def build_row(r, _):
        for j in range(_HIDDEN // _LANES):
            ds = pl.ds(j * _LANES, _LANES)
            comb0_v[r, ds] = comb0_v[r, ds] + segtab_v[0, ds]
        return _
    lax.fori_loop(0, _MAXLEN, build_row, None)
    dseg = tuple(
        segtab_v[1, pl.ds(j * _LANES, _LANES)]
        - segtab_v[0, pl.ds(j * _LANES, _LANES)]
        for j in range(_HIDDEN // _LANES)
    )

    def step_fn(s, _):
        base = wid * rows_per_w + s * _R
        pltpu.sync_copy(x_ref.at[pl.ds(base, _R)], idx_v)
        pltpu.sync_copy(seg_ref.at[pl.ds(base, _R)], sgv_v.at[pl.ds(0, _R)])
        pltpu.async_copy(tok_ref.at[idx_v], tok_v, sem).wait()

        def row_fn(r, d):
            pos = lax.rem(r, _MAXLEN)
            seg_vec = sgv_v[pl.ds(r, _LANES)]
            sf = jnp.broadcast_to(
                lax.convert_element_type(seg_vec[0], jnp.float32), (_LANES,))
            for j in range(_HIDDEN // _LANES):
                ds = pl.ds(j * _LANES, _LANES)
                tok_v[r, ds] = tok_v[r, ds] + comb0_v[pos, ds] + sf * d[j]
            return d
        lax.fori_loop(0, _R, row_fn, dseg)

        pltpu.sync_copy(tok_v, out_ref.at[pl.ds(base, _R)])
        return _
    lax.fori_loop(0, steps, step_fn, None)


def kernel(x, segment_ids, token_table, segment_table, position_table):
    batch, maxlen = x.shape
    hidden = token_table.shape[1]
    n = batch * maxlen
    x_flat = x.reshape(n)
    seg_flat = segment_ids.reshape(n)

    mesh = plsc.VectorSubcoreMesh(core_axis_name="c", subcore_axis_name="s")
    k = functools.partial(
        pl.kernel,
        mesh=mesh,
        out_type=jax.ShapeDtypeStruct((n, hidden), jnp.float32),
        scratch_types=[
            pltpu.VMEM((_R, hidden), jnp.float32),
            pltpu.VMEM((_MAXLEN, hidden), jnp.float32),
            pltpu.VMEM((_R,), jnp.int32),
            pltpu.VMEM((_R + _LANES,), jnp.int32),
            pltpu.VMEM((2, hidden), jnp.float32),
            pltpu.SemaphoreType.DMA,
        ],
    )(_body)
    out = k(x_flat, seg_flat, token_table, segment_table, position_table)
    return out.reshape(batch, maxlen, hidden)


# speedup vs baseline: 12.1228x; 4.0724x over previous
import functools
import jax
import jax.numpy as jnp
from jax import lax
from jax.experimental import pallas as pl
from jax.experimental.pallas import tpu as pltpu
from jax.experimental.pallas import tpu_sc as plsc

_HIDDEN = 128
_MAXLEN = 200
_LANES = 16
_NCORES = 2
_NSUB = 16
_NWORK = _NCORES * _NSUB
_R = _MAXLEN
_NBUF = 4


def _body(x_ref, seg_ref, tok_ref, segtab_ref, postab_ref, out_ref,
          buf0, buf1, buf2, buf3, idx0, idx1, idx2, idx3, seg_all, cidx_v,
          segtab_v, comb_sh, *sems):
    n_rows = x_ref.shape[0]
    rows_per_w = n_rows // _NWORK
    steps = rows_per_w // _R
    bufs = (buf0, buf1, buf2, buf3)
    idxs = (idx0, idx1, idx2, idx3)
    gsems, asems, ssems = sems[0:_NBUF], sems[_NBUF:2 * _NBUF], sems[2 * _NBUF:]

    cid = lax.axis_index("c")
    sid = lax.axis_index("s")
    wid = sid * _NCORES + cid
    wbase = wid * rows_per_w

    @pl.when(sid == 0)
    def _build():
        pltpu.sync_copy(postab_ref, buf0)
        pltpu.sync_copy(postab_ref, buf1)
        pltpu.sync_copy(segtab_ref, segtab_v)

        def add_seg(r, _):
            for j in range(_HIDDEN // _LANES):
                ds = pl.ds(j * _LANES, _LANES)
                buf0[r, ds] = buf0[r, ds] + segtab_v[0, ds]
                buf1[r, ds] = buf1[r, ds] + segtab_v[1, ds]
            return _
        lax.fori_loop(0, _MAXLEN, add_seg, None)
        pltpu.sync_copy(buf0, comb_sh.at[pl.ds(0, _MAXLEN)])
        pltpu.sync_copy(buf1, comb_sh.at[pl.ds(_MAXLEN, _MAXLEN)])

    plsc.subcore_barrier()

    pltpu.sync_copy(seg_ref.at[pl.ds(wbase, rows_per_w)], seg_all)

    iota = lax.iota(jnp.int32, _LANES)
    chunk_offs = [o * _LANES for o in range(_R // _LANES)] + [_R - _LANES]

    def issue_tok_gather(s):
        pltpu.sync_copy(x_ref.at[pl.ds(wbase + s * _R, _R)], idxs[s % _NBUF])
        return pltpu.async_copy(
            tok_ref.at[idxs[s % _NBUF]],
            bufs[s % _NBUF], gsems[s % _NBUF])

    gt = {}
    sc = {}
    gt[0] = issue_tok_gather(0)
    gt[1] = issue_tok_gather(1)
    for s in range(steps):
        X = bufs[s % _NBUF]
        gt[s].wait()
        for o in chunk_offs:
            sgv = seg_all[pl.ds(s * _R + o, _LANES)]
            cidx_v[pl.ds(o, _LANES)] = sgv * _MAXLEN + (iota + o)
        ga = pltpu.async_copy(comb_sh.at[cidx_v], X, asems[s % _NBUF],
                              add=True)
        if s + 2 < steps:
            if s >= 2:
                sc[s - 2].wait()
            gt[s + 2] = issue_tok_gather(s + 2)
        ga.wait()
        sc[s] = pltpu.async_copy(
            X, out_ref.at[pl.ds(wbase + s * _R, _R)], ssems[s % _NBUF])
    sc[steps - 2].wait()
    sc[steps - 1].wait()


def kernel(x, segment_ids, token_table, segment_table, position_table):
    batch, maxlen = x.shape
    hidden = token_table.shape[1]
    n = batch * maxlen
    x_flat = x.reshape(n)
    seg_flat = segment_ids.reshape(n)

    mesh = plsc.VectorSubcoreMesh(core_axis_name="c", subcore_axis_name="s")
    rows_per_w = n // _NWORK
    k = functools.partial(
        pl.kernel,
        mesh=mesh,
        out_type=jax.ShapeDtypeStruct((n, hidden), jnp.float32),
        scratch_types=[
            pltpu.VMEM((_R, hidden), jnp.float32),
            pltpu.VMEM((_R, hidden), jnp.float32),
            pltpu.VMEM((_R, hidden), jnp.float32),
            pltpu.VMEM((_R, hidden), jnp.float32),
            pltpu.VMEM((_R,), jnp.int32),
            pltpu.VMEM((_R,), jnp.int32),
            pltpu.VMEM((_R,), jnp.int32),
            pltpu.VMEM((_R,), jnp.int32),
            pltpu.VMEM((rows_per_w,), jnp.int32),
            pltpu.VMEM((_R,), jnp.int32),
            pltpu.VMEM((2, hidden), jnp.float32),
            pltpu.VMEM_SHARED((2 * _MAXLEN, hidden), jnp.float32),
        ] + [pltpu.SemaphoreType.DMA] * (3 * _NBUF),
    )(_body)
    out = k(x_flat, seg_flat, token_table, segment_table, position_table)
    return out.reshape(batch, maxlen, hidden)
